# baseline (speedup 1.0000x reference)
import functools

import jax
import jax.numpy as jnp
from jax import lax
from jax.experimental import pallas as pl
from jax.experimental.pallas import tpu as pltpu
from jax.experimental.pallas import tpu_sc as plsc

B, H, SRC, V = 1024, 128, 200, 100000
VB = 1024
NV = (V + VB - 1) // VB
BB = 16
NC, NS = 2, 16
NW = NC * NS
RPT = B // NW
CHUNKS = ((0, 104), (104, 96))

_mesh = plsc.VectorSubcoreMesh(core_axis_name="c", subcore_axis_name="s")
_sc_params = pltpu.CompilerParams(use_tc_tiling_on_sc=False)


def _wid():
    return lax.axis_index("s") * NC + lax.axis_index("c")


def _p1_body(x_ref, w_ref, b_ref, mask_ref, gm_ref, m_ref, l_ref, ms, ls):
    j = pl.program_id(0)
    gen = lax.dot_general(x_ref[...], w_ref[...], (((1,), (1,)), ((), ())),
                          preferred_element_type=jnp.float32) + b_ref[...]
    gm_ref[...] = gen - 1e6 * mask_ref[...].astype(jnp.float32)
    col = j * VB + lax.broadcasted_iota(jnp.int32, (1, VB), 1)
    s = jnp.where(col < V, gen, -1e30)
    bm = jnp.max(s, axis=1, keepdims=True)

    @pl.when(j == 0)
    def _():
        ms[...] = jnp.full((B, 1), -1e30, jnp.float32)
        ls[...] = jnp.zeros((B, 1), jnp.float32)

    mo = ms[...]
    mn = jnp.maximum(mo, bm)
    ln = ls[...] * jnp.exp(mo - mn) + jnp.sum(jnp.exp(s - mn), axis=1,
                                              keepdims=True)
    ms[...] = mn
    ls[...] = ln
    m_ref[...] = mn
    l_ref[...] = ln


def _pass1(x, W, b2, mask):
    return pl.pallas_call(
        _p1_body,
        grid=(NV,),
        in_specs=[
            pl.BlockSpec((B, H), lambda j: (0, 0)),
            pl.BlockSpec((VB, H), lambda j: (j, 0)),
            pl.BlockSpec((1, VB), lambda j: (0, j)),
            pl.BlockSpec((B, VB), lambda j: (0, j)),
        ],
        out_specs=[
            pl.BlockSpec((B, VB), lambda j: (0, j)),
            pl.BlockSpec((B, 1), lambda j: (0, 0)),
            pl.BlockSpec((B, 1), lambda j: (0, 0)),
        ],
        out_shape=[
            jax.ShapeDtypeStruct((B, V), jnp.float32),
            jax.ShapeDtypeStruct((B, 1), jnp.float32),
            jax.ShapeDtypeStruct((B, 1), jnp.float32),
        ],
        scratch_shapes=[pltpu.VMEM((B, 1), jnp.float32),
                        pltpu.VMEM((B, 1), jnp.float32)],
    )(x, W, b2, mask)


@functools.partial(
    pl.kernel,
    mesh=_mesh,
    out_type=(
        jax.ShapeDtypeStruct((B, SRC), jnp.int32),
        jax.ShapeDtypeStruct((B, SRC), jnp.float32),
        jax.ShapeDtypeStruct((B, SRC), jnp.int32),
        jax.ShapeDtypeStruct((B, SRC), jnp.float32),
    ),
    scratch_types=[
        pltpu.VMEM((RPT, SRC), jnp.int32),
        pltpu.VMEM((RPT, SRC), jnp.int32),
        pltpu.VMEM((RPT, SRC), jnp.float32),
        pltpu.VMEM((RPT, SRC), jnp.int32),
        pltpu.VMEM((RPT, SRC), jnp.float32),
        pltpu.SemaphoreType.DMA,
        pltpu.SemaphoreType.DMA,
    ],
    compiler_params=_sc_params,
)
def _sc_gather(ctx_hbm, ia_hbm, gm_hbm, mask_hbm, wsel_hbm,
               t_out, gmh_out, mh_out, wh_out,
               cvm, tvm, gvm, mvm, wvm, sem1, sem2):
    wid = _wid()
    b0 = wid * RPT
    pltpu.sync_copy(ctx_hbm.at[pl.ds(b0, RPT)], cvm)

    def fire_t(r, carry):
        for (o, n) in CHUNKS:
            pltpu.async_copy(ia_hbm.at[cvm.at[r, pl.ds(o, n)]],
                             tvm.at[r, pl.ds(o, n)], sem1)
        return carry

    lax.fori_loop(0, RPT, fire_t, 0)

    def drain_t(r, carry):
        for (o, n) in CHUNKS:
            pltpu.make_async_copy(ia_hbm.at[cvm.at[r, pl.ds(o, n)]],
                                  tvm.at[r, pl.ds(o, n)], sem1).wait()
        return carry

    lax.fori_loop(0, RPT, drain_t, 0)

    def fire_tables(r, carry):
        for (o, n) in CHUNKS:
            idx = tvm.at[r, pl.ds(o, n)]
            pltpu.async_copy(gm_hbm.at[b0 + r].at[idx],
                             gvm.at[r, pl.ds(o, n)], sem2)
            pltpu.async_copy(mask_hbm.at[b0 + r].at[idx],
                             mvm.at[r, pl.ds(o, n)], sem2)
            pltpu.async_copy(wsel_hbm.at[b0 + r].at[idx],
                             wvm.at[r, pl.ds(o, n)], sem2)
        return carry

    lax.fori_loop(0, RPT, fire_tables, 0)

    def drain_tables(r, carry):
        for (o, n) in CHUNKS:
            idx = tvm.at[r, pl.ds(o, n)]
            pltpu.make_async_copy(gm_hbm.at[b0 + r].at[idx],
                                  gvm.at[r, pl.ds(o, n)], sem2).wait()
            pltpu.make_async_copy(mask_hbm.at[b0 + r].at[idx],
                                  mvm.at[r, pl.ds(o, n)], sem2).wait()
            pltpu.make_async_copy(wsel_hbm.at[b0 + r].at[idx],
                                  wvm.at[r, pl.ds(o, n)], sem2).wait()
        return carry

    lax.fori_loop(0, RPT, drain_tables, 0)

    pltpu.sync_copy(tvm, t_out.at[pl.ds(b0, RPT)])
    pltpu.sync_copy(gvm, gmh_out.at[pl.ds(b0, RPT)])
    pltpu.sync_copy(mvm, mh_out.at[pl.ds(b0, RPT)])
    pltpu.sync_copy(wvm, wh_out.at[pl.ds(b0, RPT)])


def _corr_body(t_ref, c_ref, a_ref, gmh_ref, mh_ref, wh_ref, m0_ref, l0_ref,
               mc_ref, rc_ref, pv_ref):
    t = t_ref[...]
    val = a_ref[...] * (wh_ref[...] == c_ref[...].astype(jnp.float32)
                        ).astype(jnp.float32)
    eh = gmh_ref[...] + 1e6 * mh_ref[...].astype(jnp.float32)
    E = t[:, :, None] == t[:, None, :]
    vtot = jnp.sum(jnp.where(E, val[:, None, :], 0.0), axis=2)
    hit = eh + vtot
    s_i = lax.broadcasted_iota(jnp.int32, (1, SRC, SRC), 1)
    s_j = lax.broadcasted_iota(jnp.int32, (1, SRC, SRC), 2)
    first = ~jnp.any(E & (s_j < s_i), axis=2)
    m0 = m0_ref[...]
    l0 = l0_ref[...]
    mx = jnp.max(jnp.where(first, hit, -1e30), axis=1, keepdims=True)
    mc = jnp.maximum(m0, mx)
    zc = l0 * jnp.exp(m0 - mc) + jnp.sum(
        jnp.where(first, jnp.exp(hit - mc) - jnp.exp(eh - mc), 0.0),
        axis=1, keepdims=True)
    rc = 1.0 / zc
    mc_ref[...] = mc
    rc_ref[...] = rc
    pv_ref[...] = jnp.exp(hit - mc) * rc


def _corr(t, ctx, attn, gmh, mh, wh, M0, L0):
    nb = B // BB
    row = pl.BlockSpec((BB, SRC), lambda i: (i, 0))
    one = pl.BlockSpec((BB, 1), lambda i: (i, 0))
    return pl.pallas_call(
        _corr_body,
        grid=(nb,),
        in_specs=[row, row, row, row, row, row, one, one],
        out_specs=[one, one, row],
        out_shape=[
            jax.ShapeDtypeStruct((B, 1), jnp.float32),
            jax.ShapeDtypeStruct((B, 1), jnp.float32),
            jax.ShapeDtypeStruct((B, SRC), jnp.float32),
        ],
    )(t, ctx, attn, gmh, mh, wh, M0, L0)


def _p2_body(x_ref, w_ref, b_ref, mc_ref, rc_ref, p_ref):
    gen = lax.dot_general(x_ref[...], w_ref[...], (((1,), (1,)), ((), ())),
                          preferred_element_type=jnp.float32) + b_ref[...]
    p_ref[...] = jnp.exp(gen - mc_ref[...]) * rc_ref[...]


def _pass2(x, W, b2, Mc, Rc):
    return pl.pallas_call(
        _p2_body,
        grid=(NV,),
        in_specs=[
            pl.BlockSpec((B, H), lambda j: (0, 0)),
            pl.BlockSpec((VB, H), lambda j: (j, 0)),
            pl.BlockSpec((1, VB), lambda j: (0, j)),
            pl.BlockSpec((B, 1), lambda j: (0, 0)),
            pl.BlockSpec((B, 1), lambda j: (0, 0)),
        ],
        out_specs=pl.BlockSpec((B, VB), lambda j: (0, j)),
        out_shape=jax.ShapeDtypeStruct((B, V), jnp.float32),
    )(x, W, b2, Mc, Rc)


@functools.partial(
    pl.kernel,
    mesh=_mesh,
    out_type=(),
    scratch_types=[
        pltpu.VMEM((RPT, SRC), jnp.int32),
        pltpu.VMEM((RPT, SRC), jnp.float32),
        pltpu.SemaphoreType.DMA,
    ],
    compiler_params=_sc_params,
)
def _sc_scatter(probs_ref, t_hbm, pv_hbm, tvm, pvm, sem):
    wid = _wid()
    b0 = wid * RPT
    pltpu.sync_copy(t_hbm.at[pl.ds(b0, RPT)], tvm)
    pltpu.sync_copy(pv_hbm.at[pl.ds(b0, RPT)], pvm)

    def fire(r, carry):
        for (o, n) in CHUNKS:
            pltpu.async_copy(pvm.at[r, pl.ds(o, n)],
                             probs_ref.at[b0 + r].at[tvm.at[r, pl.ds(o, n)]],
                             sem)
        return carry

    lax.fori_loop(0, RPT, fire, 0)

    def drain(r, carry):
        for (o, n) in CHUNKS:
            pltpu.make_async_copy(pvm.at[r, pl.ds(o, n)],
                                  probs_ref.at[b0 + r].at[tvm.at[r, pl.ds(o, n)]],
                                  sem).wait()
        return carry

    lax.fori_loop(0, RPT, drain, 0)


def kernel(x, attn_scores, ctx_ids, actionmask, inp_to_act, out_map, W, b):
    ctx = ctx_ids.astype(jnp.int32)
    ia = inp_to_act.astype(jnp.int32)

    rows = jnp.arange(B, dtype=jnp.int32)[:, None]
    idxb = jnp.broadcast_to(ia[None, :], (B, ia.shape[0]))
    vals = jnp.broadcast_to(
        jnp.arange(ia.shape[0], dtype=jnp.float32)[None, :], idxb.shape)
    wsel = jnp.full((B, V), -1.0, jnp.float32).at[rows, idxb].set(vals)

    b2 = b.reshape(1, V).astype(jnp.float32)
    gm, M0, L0 = _pass1(x, W, b2, actionmask)
    t, gmh, mh, wh = _sc_gather(ctx, ia, gm, actionmask, wsel)
    Mc, Rc, pv = _corr(t, ctx, attn_scores, gmh, mh, wh, M0, L0)
    probs0 = _pass2(x, W, b2, Mc, Rc)

    probs_ref = jax.new_ref(probs0)
    _sc_scatter(probs_ref, t, pv)
    probs = jax.freeze(probs_ref)
    return (probs, gm, attn_scores)


# speedup vs baseline: 1.0021x; 1.0021x over previous
import functools

import jax
import jax.numpy as jnp
from jax import lax
from jax.experimental import pallas as pl
from jax.experimental.pallas import tpu as pltpu
from jax.experimental.pallas import tpu_sc as plsc

B, H, SRC, V = 1024, 128, 200, 100000
VB = 1024
NV = (V + VB - 1) // VB
BB = 16
NC, NS = 2, 16
NW = NC * NS
RPT = B // NW
CHUNKS = ((0, 104), (104, 96))

_mesh = plsc.VectorSubcoreMesh(core_axis_name="c", subcore_axis_name="s")
_sc_params = pltpu.CompilerParams(use_tc_tiling_on_sc=False)


def _wid():
    return lax.axis_index("s") * NC + lax.axis_index("c")


def _p1_body(x_ref, w_ref, b_ref, mask_ref, gm_ref, m_ref, l_ref, ms, ls):
    j = pl.program_id(0)
    gen = lax.dot_general(x_ref[...], w_ref[...], (((1,), (1,)), ((), ())),
                          preferred_element_type=jnp.float32) + b_ref[...]
    gm_ref[...] = gen - 1e6 * mask_ref[...].astype(jnp.float32)
    col = j * VB + lax.broadcasted_iota(jnp.int32, (1, VB), 1)
    s = jnp.where(col < V, gen, -1e30)
    bm = jnp.max(s, axis=1, keepdims=True)

    @pl.when(j == 0)
    def _():
        ms[...] = jnp.full((B, 1), -1e30, jnp.float32)
        ls[...] = jnp.zeros((B, 1), jnp.float32)

    mo = ms[...]
    mn = jnp.maximum(mo, bm)
    ln = ls[...] * jnp.exp(mo - mn) + jnp.sum(jnp.exp(s - mn), axis=1,
                                              keepdims=True)
    ms[...] = mn
    ls[...] = ln
    m_ref[...] = mn
    l_ref[...] = ln


def _pass1(x, W, b2, mask):
    return pl.pallas_call(
        _p1_body,
        grid=(NV,),
        in_specs=[
            pl.BlockSpec((B, H), lambda j: (0, 0)),
            pl.BlockSpec((VB, H), lambda j: (j, 0)),
            pl.BlockSpec((1, VB), lambda j: (0, j)),
            pl.BlockSpec((B, VB), lambda j: (0, j)),
        ],
        out_specs=[
            pl.BlockSpec((B, VB), lambda j: (0, j)),
            pl.BlockSpec((B, 1), lambda j: (0, 0)),
            pl.BlockSpec((B, 1), lambda j: (0, 0)),
        ],
        out_shape=[
            jax.ShapeDtypeStruct((B, V), jnp.float32),
            jax.ShapeDtypeStruct((B, 1), jnp.float32),
            jax.ShapeDtypeStruct((B, 1), jnp.float32),
        ],
        scratch_shapes=[pltpu.VMEM((B, 1), jnp.float32),
                        pltpu.VMEM((B, 1), jnp.float32)],
    )(x, W, b2, mask)


N = B * SRC
TPW = N // NW
NCH = TPW // 128


@functools.partial(
    pl.kernel,
    mesh=_mesh,
    out_type=(
        jax.ShapeDtypeStruct((N,), jnp.int32),
        jax.ShapeDtypeStruct((N,), jnp.float32),
        jax.ShapeDtypeStruct((N,), jnp.int32),
        jax.ShapeDtypeStruct((N,), jnp.float32),
    ),
    scratch_types=[
        pltpu.VMEM((TPW,), jnp.int32),
        pltpu.VMEM((TPW,), jnp.int32),
        pltpu.VMEM((TPW,), jnp.int32),
        pltpu.VMEM((TPW,), jnp.float32),
        pltpu.VMEM((TPW,), jnp.int32),
        pltpu.VMEM((TPW,), jnp.float32),
        pltpu.SemaphoreType.DMA,
        pltpu.SemaphoreType.DMA,
    ],
    compiler_params=_sc_params,
)
def _sc_gather(ctx_hbm, rowf_hbm, ia_hbm, gm_hbm, mask_hbm, wsel_hbm,
               t_out, gmh_out, mh_out, wh_out,
               cvm, tvm, wix, gvm, mvm, wvm, sem1, sem2):
    wid = _wid()
    b0 = wid * RPT
    pltpu.sync_copy(ctx_hbm.at[pl.ds(b0 * SRC, TPW)], cvm)

    def fire_t(r, carry):
        for (o, n) in CHUNKS:
            pltpu.async_copy(ia_hbm.at[cvm.at[pl.ds(r * SRC + o, n)]],
                             tvm.at[pl.ds(r * SRC + o, n)], sem1)
        return carry

    lax.fori_loop(0, RPT, fire_t, 0)

    def drain_t(r, carry):
        for (o, n) in CHUNKS:
            pltpu.make_async_copy(ia_hbm.at[cvm.at[pl.ds(r * SRC + o, n)]],
                                  tvm.at[pl.ds(r * SRC + o, n)], sem1).wait()
        return carry

    lax.fori_loop(0, RPT, drain_t, 0)

    pltpu.sync_copy(rowf_hbm.at[pl.ds(b0 * SRC, TPW)], wix)

    def mk_widx(k, carry):
        sl = pl.ds(k * 16, 16)
        wix[sl] = tvm[sl] * B + wix[sl]
        return carry

    lax.fori_loop(0, TPW // 16, mk_widx, 0)

    def fire_tables(r, carry):
        for (o, n) in CHUNKS:
            idx = tvm.at[pl.ds(r * SRC + o, n)]
            pltpu.async_copy(gm_hbm.at[b0 + r].at[idx],
                             gvm.at[pl.ds(r * SRC + o, n)], sem2)
            pltpu.async_copy(mask_hbm.at[b0 + r].at[idx],
                             mvm.at[pl.ds(r * SRC + o, n)], sem2)
        return carry

    lax.fori_loop(0, RPT, fire_tables, 0)

    def fire_wsel(j, carry):
        pltpu.async_copy(wsel_hbm.at[wix.at[pl.ds(j * 128, 128)]],
                         wvm.at[pl.ds(j * 128, 128)], sem1)
        return carry

    lax.fori_loop(0, NCH, fire_wsel, 0)

    def drain_tables(r, carry):
        for (o, n) in CHUNKS:
            idx = tvm.at[pl.ds(r * SRC + o, n)]
            pltpu.make_async_copy(gm_hbm.at[b0 + r].at[idx],
                                  gvm.at[pl.ds(r * SRC + o, n)], sem2).wait()
            pltpu.make_async_copy(mask_hbm.at[b0 + r].at[idx],
                                  mvm.at[pl.ds(r * SRC + o, n)], sem2).wait()
        return carry

    lax.fori_loop(0, RPT, drain_tables, 0)

    def drain_wsel(j, carry):
        pltpu.make_async_copy(wsel_hbm.at[wix.at[pl.ds(j * 128, 128)]],
                              wvm.at[pl.ds(j * 128, 128)], sem1).wait()
        return carry

    lax.fori_loop(0, NCH, drain_wsel, 0)

    pltpu.sync_copy(tvm, t_out.at[pl.ds(b0 * SRC, TPW)])
    pltpu.sync_copy(gvm, gmh_out.at[pl.ds(b0 * SRC, TPW)])
    pltpu.sync_copy(mvm, mh_out.at[pl.ds(b0 * SRC, TPW)])
    pltpu.sync_copy(wvm, wh_out.at[pl.ds(b0 * SRC, TPW)])


def _corr_body(t_ref, c_ref, a_ref, gmh_ref, mh_ref, wh_ref, m0_ref, l0_ref,
               mc_ref, rc_ref, pv_ref):
    t = t_ref[...]
    val = a_ref[...] * (wh_ref[...] == c_ref[...].astype(jnp.float32)
                        ).astype(jnp.float32)
    eh = gmh_ref[...] + 1e6 * mh_ref[...].astype(jnp.float32)
    E = t[:, :, None] == t[:, None, :]
    vtot = jnp.sum(jnp.where(E, val[:, None, :], 0.0), axis=2)
    hit = eh + vtot
    s_i = lax.broadcasted_iota(jnp.int32, (1, SRC, SRC), 1)
    s_j = lax.broadcasted_iota(jnp.int32, (1, SRC, SRC), 2)
    first = ~jnp.any(E & (s_j < s_i), axis=2)
    m0 = m0_ref[...]
    l0 = l0_ref[...]
    mx = jnp.max(jnp.where(first, hit, -1e30), axis=1, keepdims=True)
    mc = jnp.maximum(m0, mx)
    zc = l0 * jnp.exp(m0 - mc) + jnp.sum(
        jnp.where(first, jnp.exp(hit - mc) - jnp.exp(eh - mc), 0.0),
        axis=1, keepdims=True)
    rc = 1.0 / zc
    mc_ref[...] = mc
    rc_ref[...] = rc
    pv_ref[...] = jnp.exp(hit - mc) * rc


def _corr(t, ctx, attn, gmh, mh, wh, M0, L0):
    nb = B // BB
    row = pl.BlockSpec((BB, SRC), lambda i: (i, 0))
    one = pl.BlockSpec((BB, 1), lambda i: (i, 0))
    return pl.pallas_call(
        _corr_body,
        grid=(nb,),
        in_specs=[row, row, row, row, row, row, one, one],
        out_specs=[one, one, row],
        out_shape=[
            jax.ShapeDtypeStruct((B, 1), jnp.float32),
            jax.ShapeDtypeStruct((B, 1), jnp.float32),
            jax.ShapeDtypeStruct((B, SRC), jnp.float32),
        ],
    )(t, ctx, attn, gmh, mh, wh, M0, L0)


def _p2_body(x_ref, w_ref, b_ref, mc_ref, rc_ref, p_ref):
    gen = lax.dot_general(x_ref[...], w_ref[...], (((1,), (1,)), ((), ())),
                          preferred_element_type=jnp.float32) + b_ref[...]
    p_ref[...] = jnp.exp(gen - mc_ref[...]) * rc_ref[...]


def _pass2(x, W, b2, Mc, Rc):
    return pl.pallas_call(
        _p2_body,
        grid=(NV,),
        in_specs=[
            pl.BlockSpec((B, H), lambda j: (0, 0)),
            pl.BlockSpec((VB, H), lambda j: (j, 0)),
            pl.BlockSpec((1, VB), lambda j: (0, j)),
            pl.BlockSpec((B, 1), lambda j: (0, 0)),
            pl.BlockSpec((B, 1), lambda j: (0, 0)),
        ],
        out_specs=pl.BlockSpec((B, VB), lambda j: (0, j)),
        out_shape=jax.ShapeDtypeStruct((B, V), jnp.float32),
    )(x, W, b2, Mc, Rc)


@functools.partial(
    pl.kernel,
    mesh=_mesh,
    out_type=(),
    scratch_types=[
        pltpu.VMEM((RPT, SRC), jnp.int32),
        pltpu.VMEM((RPT, SRC), jnp.float32),
        pltpu.SemaphoreType.DMA,
    ],
    compiler_params=_sc_params,
)
def _sc_scatter(probs_ref, t_hbm, pv_hbm, tvm, pvm, sem):
    wid = _wid()
    b0 = wid * RPT
    pltpu.sync_copy(t_hbm.at[pl.ds(b0, RPT)], tvm)
    pltpu.sync_copy(pv_hbm.at[pl.ds(b0, RPT)], pvm)

    def fire(r, carry):
        for (o, n) in CHUNKS:
            pltpu.async_copy(pvm.at[r, pl.ds(o, n)],
                             probs_ref.at[b0 + r].at[tvm.at[r, pl.ds(o, n)]],
                             sem)
        return carry

    lax.fori_loop(0, RPT, fire, 0)

    def drain(r, carry):
        for (o, n) in CHUNKS:
            pltpu.make_async_copy(pvm.at[r, pl.ds(o, n)],
                                  probs_ref.at[b0 + r].at[tvm.at[r, pl.ds(o, n)]],
                                  sem).wait()
        return carry

    lax.fori_loop(0, RPT, drain, 0)


def kernel(x, attn_scores, ctx_ids, actionmask, inp_to_act, out_map, W, b):
    ctx = ctx_ids.astype(jnp.int32)
    ia = inp_to_act.astype(jnp.int32)

    VI = ia.shape[0]
    rows = jnp.arange(B, dtype=jnp.int32)[:, None]
    idxb = jnp.broadcast_to(ia[None, :], (B, VI))
    vals = jnp.broadcast_to(
        jnp.arange(VI, dtype=jnp.float32)[None, :], (B, VI))
    wsel = jnp.zeros((B, V), jnp.float32).at[rows, idxb].set(vals)
    wsel_flat = wsel.T.reshape(-1)

    b2 = b.reshape(1, V).astype(jnp.float32)
    gm, M0, L0 = _pass1(x, W, b2, actionmask)
    rowf = jnp.arange(N, dtype=jnp.int32) // SRC
    tf, gmhf, mhf, whf = _sc_gather(ctx.reshape(-1), rowf, ia, gm,
                                    actionmask, wsel_flat)
    t = tf.reshape(B, SRC)
    Mc, Rc, pv = _corr(t, ctx, attn_scores, gmhf.reshape(B, SRC),
                       mhf.reshape(B, SRC), whf.reshape(B, SRC), M0, L0)
    probs0 = _pass2(x, W, b2, Mc, Rc)

    probs_ref = jax.new_ref(probs0)
    _sc_scatter(probs_ref, t, pv)
    probs = jax.freeze(probs_ref)
    return (probs, gm, attn_scores)
